# baseline (speedup 1.0000x reference)
import functools

import jax
import jax.numpy as jnp
from jax import lax
from jax.experimental import pallas as pl
from jax.experimental.pallas import tpu as pltpu
from jax.experimental.pallas import tpu_sc as plsc

N_NODES = 10000
N_EDGES = 320000
D = 128
D_EDGE = 16
NG = 16
NOUT_G = 4

NC = 2
NS = 16
NW = NC * NS
E_PER_W = N_EDGES // NW
K = 40
NBLK = E_PER_W // K
SUP = 50
KSUP = K * SUP
STRIPE = 624
REM = N_NODES - NS * STRIPE


def _ab_body(x_ref, ws_ref, wr_ref, a_ref, b_ref):
    x = x_ref[...]
    a_ref[...] = jnp.dot(x, ws_ref[...], preferred_element_type=jnp.float32)
    b_ref[...] = jnp.dot(x, wr_ref[...], preferred_element_type=jnp.float32)


def _make_ab(rows_blk):
    nb = N_NODES // rows_blk
    return pl.pallas_call(
        _ab_body,
        grid=(nb,),
        in_specs=[
            pl.BlockSpec((rows_blk, D), lambda i: (i, 0)),
            pl.BlockSpec((D, D), lambda i: (0, 0)),
            pl.BlockSpec((D, D), lambda i: (0, 0)),
        ],
        out_specs=[
            pl.BlockSpec((rows_blk, D), lambda i: (i, 0)),
            pl.BlockSpec((rows_blk, D), lambda i: (i, 0)),
        ],
        out_shape=[
            jax.ShapeDtypeStruct((N_NODES, D), jnp.float32),
            jax.ShapeDtypeStruct((N_NODES, D), jnp.float32),
        ],
    )


def _c_body(ef_ref, we_ref, be_ref, c_ref):
    c_ref[...] = (
        jnp.dot(ef_ref[...], we_ref[...], preferred_element_type=jnp.float32)
        + be_ref[...]
    )


def _make_c(edge_blk):
    nb = N_EDGES // edge_blk
    return pl.pallas_call(
        _c_body,
        grid=(nb,),
        in_specs=[
            pl.BlockSpec((edge_blk, D_EDGE), lambda i: (i, 0)),
            pl.BlockSpec((D_EDGE, D), lambda i: (0, 0)),
            pl.BlockSpec((1, D), lambda i: (0, 0)),
        ],
        out_specs=pl.BlockSpec((edge_blk, D), lambda i: (i, 0)),
        out_shape=jax.ShapeDtypeStruct((N_EDGES, D), jnp.float32),
    )


def _sc_body(recv_hbm, send_hbm, a_hbm, b_hbm, c_hbm, out_hbm,
             sr0, sr1, rsup, ssup,
             ar0, ar1, br0, br1, cr0, cr1, mg0, mg1,
             agg_sh, sga0, sga1, ssc0, ssc1):
    cid = lax.axis_index("c")
    sid = lax.axis_index("s")
    wid = sid * NC + cid
    srx = (sr0, sr1)
    ar = (ar0, ar1)
    br = (br0, br1)
    cr = (cr0, cr1)
    mg = (mg0, mg1)
    sga = (sga0, sga1)
    ssc = (ssc0, ssc1)

    def zrow(r, carry):
        for k8 in range(D // 16):
            mg0[r, pl.ds(k8 * 16, 16)] = jnp.zeros((16,), jnp.float32)
        return carry

    lax.fori_loop(0, K, zrow, 0)
    for j in range(STRIPE // K):
        pltpu.sync_copy(mg0, agg_sh.at[pl.ds(sid * STRIPE + j * K, K)])
    pltpu.sync_copy(
        mg0.at[pl.ds(0, STRIPE % K)],
        agg_sh.at[pl.ds(sid * STRIPE + (STRIPE // K) * K, STRIPE % K)],
    )

    @pl.when(sid == 0)
    def _():
        pltpu.sync_copy(mg0.at[pl.ds(0, REM)],
                        agg_sh.at[pl.ds(NS * STRIPE, REM)])

    plsc.subcore_barrier()

    ebase = wid * E_PER_W

    def issue_gathers(b, jnext):
        off = jnext % SUP
        pltpu.async_copy(
            a_hbm.at[ssup.at[pl.ds(off * K, K)]], ar[b], sga[b]
        )
        pltpu.async_copy(
            b_hbm.at[rsup.at[pl.ds(off * K, K)]], br[b], sga[b]
        )
        pltpu.async_copy(
            c_hbm.at[pl.ds(ebase + jnext * K, K)], cr[b], sga[b]
        )

    def sub_iter(jj, b):
        pltpu.make_async_copy(a_hbm.at[pl.ds(0, K)], ar[b], sga[b]).wait()
        pltpu.make_async_copy(b_hbm.at[pl.ds(0, K)], br[b], sga[b]).wait()
        pltpu.make_async_copy(c_hbm.at[pl.ds(0, K)], cr[b], sga[b]).wait()

        @pl.when(jj >= 2)
        def _():
            pltpu.make_async_copy(mg[b], agg_sh.at[pl.ds(0, K)],
                                  ssc[b]).wait()

        off = (jj % SUP) * K
        srx[b][pl.ds(0, 16)] = rsup[pl.ds(off, 16)]
        srx[b][pl.ds(16, 16)] = rsup[pl.ds(off + 16, 16)]
        srx[b][pl.ds(K - 16, 16)] = rsup[pl.ds(off + K - 16, 16)]

        @pl.when(jj + 1 < NBLK)
        def _():
            @pl.when((jj + 1) % SUP == 0)
            def _():
                nb = ebase + (jj + 1) * K
                pltpu.sync_copy(recv_hbm.at[pl.ds(nb, KSUP)], rsup)
                pltpu.sync_copy(send_hbm.at[pl.ds(nb, KSUP)], ssup)

            issue_gathers(1 - b, jj + 1)

        def row(r, c2):
            for k8 in range(D // 16):
                sl = pl.ds(k8 * 16, 16)
                mg[b][r, sl] = jnp.maximum(
                    ar[b][r, sl] + br[b][r, sl] + cr[b][r, sl], 0.0
                )
            return c2

        lax.fori_loop(0, K, row, 0, unroll=4)
        pltpu.async_copy(mg[b], agg_sh.at[srx[b]], ssc[b], add=True)

    pltpu.sync_copy(recv_hbm.at[pl.ds(ebase, KSUP)], rsup)
    pltpu.sync_copy(send_hbm.at[pl.ds(ebase, KSUP)], ssup)
    issue_gathers(0, 0)

    def pair(p, carry):
        sub_iter(2 * p, 0)
        sub_iter(2 * p + 1, 1)
        return carry

    lax.fori_loop(0, NBLK // 2, pair, 0)
    pltpu.make_async_copy(mg0, agg_sh.at[pl.ds(0, K)], ssc0).wait()
    pltpu.make_async_copy(mg1, agg_sh.at[pl.ds(0, K)], ssc1).wait()
    plsc.subcore_barrier()
    pltpu.sync_copy(
        agg_sh.at[pl.ds(sid * STRIPE, STRIPE)],
        out_hbm.at[cid].at[pl.ds(sid * STRIPE, STRIPE)],
    )

    @pl.when(sid == 0)
    def _():
        pltpu.sync_copy(
            agg_sh.at[pl.ds(NS * STRIPE, REM)],
            out_hbm.at[cid].at[pl.ds(NS * STRIPE, REM)],
        )


@functools.cache
def _get_sc_agg():
    return pl.kernel(
        _sc_body,
        out_type=jax.ShapeDtypeStruct((NC, N_NODES, D), jnp.float32),
        mesh=plsc.VectorSubcoreMesh(
            core_axis_name="c", subcore_axis_name="s",
            num_cores=NC, num_subcores=NS,
        ),
        scratch_types=[
            pltpu.VMEM((K,), jnp.int32),
            pltpu.VMEM((K,), jnp.int32),
            pltpu.VMEM((KSUP,), jnp.int32),
            pltpu.VMEM((KSUP,), jnp.int32),
            pltpu.VMEM((K, D), jnp.float32),
            pltpu.VMEM((K, D), jnp.float32),
            pltpu.VMEM((K, D), jnp.float32),
            pltpu.VMEM((K, D), jnp.float32),
            pltpu.VMEM((K, D), jnp.float32),
            pltpu.VMEM((K, D), jnp.float32),
            pltpu.VMEM((K, D), jnp.float32),
            pltpu.VMEM((K, D), jnp.float32),
            pltpu.VMEM_SHARED((N_NODES, D), jnp.float32),
            pltpu.SemaphoreType.DMA,
            pltpu.SemaphoreType.DMA,
            pltpu.SemaphoreType.DMA,
            pltpu.SemaphoreType.DMA,
        ],
    )


def _node_body(x_ref, g0_ref, g1_ref, batch_ref, wxn_ref, wan_ref, bn_ref,
               wg5_ref, bg5_ref, node_ref, glob_ref, acc_ref):
    i = pl.program_id(0)
    nsteps = pl.num_programs(0)
    x = x_ref[...]
    ag = g0_ref[...] + g1_ref[...]
    node_ref[...] = (
        jnp.dot(x, wxn_ref[...], preferred_element_type=jnp.float32)
        + jnp.dot(ag, wan_ref[...], preferred_element_type=jnp.float32)
        + bn_ref[...]
    )
    g5 = (
        jnp.dot(x, wg5_ref[: D, :], preferred_element_type=jnp.float32)
        + jnp.dot(ag, wg5_ref[D:, :], preferred_element_type=jnp.float32)
        + bg5_ref[...]
    )
    b = batch_ref[0, 0, :]
    onehot = (
        b[:, None] == lax.broadcasted_iota(jnp.int32, (1, NG), 1)
    ).astype(jnp.float32)
    part = lax.dot_general(
        onehot, g5, (((0,), (0,)), ((), ())),
        preferred_element_type=jnp.float32,
    )

    @pl.when(i == 0)
    def _():
        acc_ref[...] = part

    @pl.when(i > 0)
    def _():
        acc_ref[...] = acc_ref[...] + part

    @pl.when(i == nsteps - 1)
    def _():
        acc = acc_ref[...]
        glob_ref[...] = acc[:, :NOUT_G] / jnp.maximum(acc[:, NOUT_G:NOUT_G + 1], 1.0)


def _make_node(rows_blk):
    nb = N_NODES // rows_blk
    return pl.pallas_call(
        _node_body,
        grid=(nb,),
        in_specs=[
            pl.BlockSpec((rows_blk, D), lambda i: (i, 0)),
            pl.BlockSpec((rows_blk, D), lambda i: (i, 0)),
            pl.BlockSpec((rows_blk, D), lambda i: (i, 0)),
            pl.BlockSpec((1, 1, rows_blk), lambda i: (i, 0, 0)),
            pl.BlockSpec((D, D), lambda i: (0, 0)),
            pl.BlockSpec((D, D), lambda i: (0, 0)),
            pl.BlockSpec((1, D), lambda i: (0, 0)),
            pl.BlockSpec((2 * D, NOUT_G + 1), lambda i: (0, 0)),
            pl.BlockSpec((1, NOUT_G + 1), lambda i: (0, 0)),
        ],
        out_specs=[
            pl.BlockSpec((rows_blk, D), lambda i: (i, 0)),
            pl.BlockSpec((NG, NOUT_G), lambda i: (0, 0)),
        ],
        out_shape=[
            jax.ShapeDtypeStruct((N_NODES, D), jnp.float32),
            jax.ShapeDtypeStruct((NG, NOUT_G), jnp.float32),
        ],
        scratch_shapes=[pltpu.VMEM((NG, NOUT_G + 1), jnp.float32)],
    )


@jax.jit
def kernel(x, edge_index, edge_features, batch, W_e, b_e, W_n, b_n):
    receivers = edge_index[0]
    senders = edge_index[1]

    a, b = _make_ab(2000)(x, W_e[:D, :], W_e[D:2 * D, :])
    c = _make_c(4000)(edge_features, W_e[2 * D:, :], b_e.reshape(1, D))

    agg = _get_sc_agg()(receivers, senders, a, b, c)

    wg5 = jnp.zeros((2 * D, NOUT_G + 1), jnp.float32).at[:, :NOUT_G].set(
        W_n[:, :NOUT_G]
    )
    bg5 = jnp.zeros((1, NOUT_G + 1), jnp.float32).at[0, :NOUT_G].set(
        b_n[:NOUT_G]
    ).at[0, NOUT_G].set(1.0)

    node_out, global_output = _make_node(2000)(
        x, agg[0], agg[1], batch.reshape(5, 1, 2000),
        W_n[:D, NOUT_G:], W_n[D:, NOUT_G:], b_n[NOUT_G:].reshape(1, D),
        wg5, bg5,
    )
    return node_out, global_output


# speedup vs baseline: 1.3588x; 1.3588x over previous
import functools

import jax
import jax.numpy as jnp
from jax import lax
from jax.experimental import pallas as pl
from jax.experimental.pallas import tpu as pltpu
from jax.experimental.pallas import tpu_sc as plsc

N_NODES = 10000
N_EDGES = 320000
D = 128
D_EDGE = 16
NG = 16
NOUT_G = 4

NC = 2
NS = 16
NW = NC * NS
E_PER_W = N_EDGES // NW
K = 40
NBLK = E_PER_W // K
SUP = 50
KSUP = K * SUP
STRIPE = 624
REM = N_NODES - NS * STRIPE


def _ab_body(x_ref, ws_ref, wr_ref, a_ref, b_ref):
    x = x_ref[...]
    a_ref[...] = jnp.dot(x, ws_ref[...], preferred_element_type=jnp.float32)
    b_ref[...] = jnp.dot(x, wr_ref[...], preferred_element_type=jnp.float32)


def _make_ab(rows_blk):
    nb = N_NODES // rows_blk
    return pl.pallas_call(
        _ab_body,
        grid=(nb,),
        in_specs=[
            pl.BlockSpec((rows_blk, D), lambda i: (i, 0)),
            pl.BlockSpec((D, D), lambda i: (0, 0)),
            pl.BlockSpec((D, D), lambda i: (0, 0)),
        ],
        out_specs=[
            pl.BlockSpec((rows_blk, D), lambda i: (i, 0)),
            pl.BlockSpec((rows_blk, D), lambda i: (i, 0)),
        ],
        out_shape=[
            jax.ShapeDtypeStruct((N_NODES, D), jnp.float32),
            jax.ShapeDtypeStruct((N_NODES, D), jnp.float32),
        ],
    )


def _c_body(ef_ref, we_ref, be_ref, c_ref):
    c_ref[...] = (
        jnp.dot(ef_ref[...], we_ref[...], preferred_element_type=jnp.float32)
        + be_ref[...]
    )


def _make_c(edge_blk):
    nb = N_EDGES // edge_blk
    return pl.pallas_call(
        _c_body,
        grid=(nb,),
        in_specs=[
            pl.BlockSpec((edge_blk, D_EDGE), lambda i: (i, 0)),
            pl.BlockSpec((D_EDGE, D), lambda i: (0, 0)),
            pl.BlockSpec((1, D), lambda i: (0, 0)),
        ],
        out_specs=pl.BlockSpec((edge_blk, D), lambda i: (i, 0)),
        out_shape=jax.ShapeDtypeStruct((N_EDGES, D), jnp.float32),
    )


def _sc_body(recv_hbm, send_hbm, a_hbm, b_hbm, c_hbm, out_hbm,
             sr0, sr1, rsup, ssup,
             ar0, ar1, br0, br1, cr0, cr1, mg0, mg1,
             agg_sh, sga0, sga1, ssc0, ssc1):
    cid = lax.axis_index("c")
    sid = lax.axis_index("s")
    wid = sid * NC + cid
    srx = (sr0, sr1)
    ar = (ar0, ar1)
    br = (br0, br1)
    cr = (cr0, cr1)
    mg = (mg0, mg1)
    sga = (sga0, sga1)
    ssc = (ssc0, ssc1)

    def zrow(r, carry):
        for k8 in range(D // 16):
            mg0[r, pl.ds(k8 * 16, 16)] = jnp.zeros((16,), jnp.float32)
        return carry

    lax.fori_loop(0, K, zrow, 0)
    for j in range(STRIPE // K):
        pltpu.sync_copy(mg0, agg_sh.at[pl.ds(sid * STRIPE + j * K, K)])
    pltpu.sync_copy(
        mg0.at[pl.ds(0, STRIPE % K)],
        agg_sh.at[pl.ds(sid * STRIPE + (STRIPE // K) * K, STRIPE % K)],
    )

    @pl.when(sid == 0)
    def _():
        pltpu.sync_copy(mg0.at[pl.ds(0, REM)],
                        agg_sh.at[pl.ds(NS * STRIPE, REM)])

    plsc.subcore_barrier()

    ebase = wid * E_PER_W

    def issue_gathers(b, jnext):
        off = jnext % SUP
        pltpu.async_copy(
            a_hbm.at[ssup.at[pl.ds(off * K, K)]], ar[b], sga[b]
        )
        pltpu.async_copy(
            b_hbm.at[rsup.at[pl.ds(off * K, K)]], br[b], sga[b]
        )
        pltpu.async_copy(
            c_hbm.at[pl.ds(ebase + jnext * K, K)], cr[b], sga[b]
        )

    def sub_iter(jj, b):
        pltpu.make_async_copy(a_hbm.at[pl.ds(0, K)], ar[b], sga[b]).wait()
        pltpu.make_async_copy(b_hbm.at[pl.ds(0, K)], br[b], sga[b]).wait()
        pltpu.make_async_copy(c_hbm.at[pl.ds(0, K)], cr[b], sga[b]).wait()

        @pl.when(jj >= 2)
        def _():
            pltpu.make_async_copy(mg[b], agg_sh.at[pl.ds(0, K)],
                                  ssc[b]).wait()

        off = (jj % SUP) * K
        srx[b][pl.ds(0, 16)] = rsup[pl.ds(off, 16)]
        srx[b][pl.ds(16, 16)] = rsup[pl.ds(off + 16, 16)]
        srx[b][pl.ds(K - 16, 16)] = rsup[pl.ds(off + K - 16, 16)]

        @pl.when(jj + 1 < NBLK)
        def _():
            @pl.when((jj + 1) % SUP == 0)
            def _():
                nb = ebase + (jj + 1) * K
                pltpu.sync_copy(recv_hbm.at[pl.ds(nb, KSUP)], rsup)
                pltpu.sync_copy(send_hbm.at[pl.ds(nb, KSUP)], ssup)

            issue_gathers(1 - b, jj + 1)

        def row(r, c2):
            for k8 in range(D // 16):
                sl = pl.ds(k8 * 16, 16)
                mg[b][r, sl] = jnp.maximum(
                    ar[b][r, sl] + br[b][r, sl] + cr[b][r, sl], 0.0
                )
            return c2

        lax.fori_loop(0, K, row, 0)
        pltpu.async_copy(mg[b], agg_sh.at[srx[b]], ssc[b], add=True)

    pltpu.sync_copy(recv_hbm.at[pl.ds(ebase, KSUP)], rsup)
    pltpu.sync_copy(send_hbm.at[pl.ds(ebase, KSUP)], ssup)
    issue_gathers(0, 0)

    def pair(p, carry):
        sub_iter(2 * p, 0)
        sub_iter(2 * p + 1, 1)
        return carry

    lax.fori_loop(0, NBLK // 2, pair, 0)
    pltpu.make_async_copy(mg0, agg_sh.at[pl.ds(0, K)], ssc0).wait()
    pltpu.make_async_copy(mg1, agg_sh.at[pl.ds(0, K)], ssc1).wait()
    plsc.subcore_barrier()
    pltpu.sync_copy(
        agg_sh.at[pl.ds(sid * STRIPE, STRIPE)],
        out_hbm.at[cid].at[pl.ds(sid * STRIPE, STRIPE)],
    )

    @pl.when(sid == 0)
    def _():
        pltpu.sync_copy(
            agg_sh.at[pl.ds(NS * STRIPE, REM)],
            out_hbm.at[cid].at[pl.ds(NS * STRIPE, REM)],
        )


@functools.cache
def _get_sc_agg():
    return pl.kernel(
        _sc_body,
        out_type=jax.ShapeDtypeStruct((NC, N_NODES, D), jnp.float32),
        mesh=plsc.VectorSubcoreMesh(
            core_axis_name="c", subcore_axis_name="s",
            num_cores=NC, num_subcores=NS,
        ),
        scratch_types=[
            pltpu.VMEM((K,), jnp.int32),
            pltpu.VMEM((K,), jnp.int32),
            pltpu.VMEM((KSUP,), jnp.int32),
            pltpu.VMEM((KSUP,), jnp.int32),
            pltpu.VMEM((K, D), jnp.float32),
            pltpu.VMEM((K, D), jnp.float32),
            pltpu.VMEM((K, D), jnp.float32),
            pltpu.VMEM((K, D), jnp.float32),
            pltpu.VMEM((K, D), jnp.float32),
            pltpu.VMEM((K, D), jnp.float32),
            pltpu.VMEM((K, D), jnp.float32),
            pltpu.VMEM((K, D), jnp.float32),
            pltpu.VMEM_SHARED((N_NODES, D), jnp.float32),
            pltpu.SemaphoreType.DMA,
            pltpu.SemaphoreType.DMA,
            pltpu.SemaphoreType.DMA,
            pltpu.SemaphoreType.DMA,
        ],
    )


def _node_body(x_ref, g0_ref, g1_ref, batch_ref, wxn_ref, wan_ref, bn_ref,
               wg5_ref, bg5_ref, node_ref, glob_ref, acc_ref):
    i = pl.program_id(0)
    nsteps = pl.num_programs(0)
    x = x_ref[...]
    ag = g0_ref[...] + g1_ref[...]
    node_ref[...] = (
        jnp.dot(x, wxn_ref[...], preferred_element_type=jnp.float32)
        + jnp.dot(ag, wan_ref[...], preferred_element_type=jnp.float32)
        + bn_ref[...]
    )
    g5 = (
        jnp.dot(x, wg5_ref[: D, :], preferred_element_type=jnp.float32)
        + jnp.dot(ag, wg5_ref[D:, :], preferred_element_type=jnp.float32)
        + bg5_ref[...]
    )
    b = batch_ref[0, 0, :]
    onehot = (
        b[:, None] == lax.broadcasted_iota(jnp.int32, (1, NG), 1)
    ).astype(jnp.float32)
    part = lax.dot_general(
        onehot, g5, (((0,), (0,)), ((), ())),
        preferred_element_type=jnp.float32,
    )

    @pl.when(i == 0)
    def _():
        acc_ref[...] = part

    @pl.when(i > 0)
    def _():
        acc_ref[...] = acc_ref[...] + part

    @pl.when(i == nsteps - 1)
    def _():
        acc = acc_ref[...]
        glob_ref[...] = acc[:, :NOUT_G] / jnp.maximum(acc[:, NOUT_G:NOUT_G + 1], 1.0)


def _make_node(rows_blk):
    nb = N_NODES // rows_blk
    return pl.pallas_call(
        _node_body,
        grid=(nb,),
        in_specs=[
            pl.BlockSpec((rows_blk, D), lambda i: (i, 0)),
            pl.BlockSpec((rows_blk, D), lambda i: (i, 0)),
            pl.BlockSpec((rows_blk, D), lambda i: (i, 0)),
            pl.BlockSpec((1, 1, rows_blk), lambda i: (i, 0, 0)),
            pl.BlockSpec((D, D), lambda i: (0, 0)),
            pl.BlockSpec((D, D), lambda i: (0, 0)),
            pl.BlockSpec((1, D), lambda i: (0, 0)),
            pl.BlockSpec((2 * D, NOUT_G + 1), lambda i: (0, 0)),
            pl.BlockSpec((1, NOUT_G + 1), lambda i: (0, 0)),
        ],
        out_specs=[
            pl.BlockSpec((rows_blk, D), lambda i: (i, 0)),
            pl.BlockSpec((NG, NOUT_G), lambda i: (0, 0)),
        ],
        out_shape=[
            jax.ShapeDtypeStruct((N_NODES, D), jnp.float32),
            jax.ShapeDtypeStruct((NG, NOUT_G), jnp.float32),
        ],
        scratch_shapes=[pltpu.VMEM((NG, NOUT_G + 1), jnp.float32)],
    )


@jax.jit
def kernel(x, edge_index, edge_features, batch, W_e, b_e, W_n, b_n):
    receivers = edge_index[0]
    senders = edge_index[1]

    a, b = _make_ab(2000)(x, W_e[:D, :], W_e[D:2 * D, :])
    c = _make_c(4000)(edge_features, W_e[2 * D:, :], b_e.reshape(1, D))

    agg = _get_sc_agg()(receivers, senders, a, b, c)

    wg5 = jnp.zeros((2 * D, NOUT_G + 1), jnp.float32).at[:, :NOUT_G].set(
        W_n[:, :NOUT_G]
    )
    bg5 = jnp.zeros((1, NOUT_G + 1), jnp.float32).at[0, :NOUT_G].set(
        b_n[:NOUT_G]
    ).at[0, NOUT_G].set(1.0)

    node_out, global_output = _make_node(2000)(
        x, agg[0], agg[1], batch.reshape(5, 1, 2000),
        W_n[:D, NOUT_G:], W_n[D:, NOUT_G:], b_n[NOUT_G:].reshape(1, D),
        wg5, bg5,
    )
    return node_out, global_output
